# baseline (speedup 1.0000x reference)
import functools

import jax
import jax.numpy as jnp
from jax import lax
from jax.experimental import pallas as pl
from jax.experimental.pallas import tpu as pltpu
from jax.experimental.pallas import tpu_sc as plsc

VOCAB = 100000
EMBED_DIM = 32
BATCH = 1024
CTX = 20

NC = 2
NS = 16
NW = NC * NS
B_PER_W = BATCH // NW
IDX_PER_W = B_PER_W * CTX
IDX_CHUNK = 128
N_CHUNKS = IDX_PER_W // IDX_CHUNK


def _pool_body(idx_hbm, emb_hbm, out_hbm, idx_v, rows_v, pooled_v, sem):
    c = lax.axis_index("c")
    s = lax.axis_index("s")
    wid = s * NC + c

    pltpu.sync_copy(idx_hbm.at[wid], idx_v)

    cps = [
        pltpu.async_copy(
            emb_hbm.at[idx_v.at[j]],
            rows_v.at[pl.ds(j * IDX_CHUNK, IDX_CHUNK)],
            sem,
        )
        for j in range(N_CHUNKS)
    ]
    for cp in cps:
        cp.wait()

    def row_body(b, carry):
        base = b * CTX
        acc0 = jnp.zeros((16,), jnp.float32)
        acc1 = jnp.zeros((16,), jnp.float32)
        for k in range(CTX):
            acc0 = acc0 + rows_v[base + k, pl.ds(0, 16)]
            acc1 = acc1 + rows_v[base + k, pl.ds(16, 16)]
        pooled_v[b, pl.ds(0, 16)] = acc0 * (1.0 / CTX)
        pooled_v[b, pl.ds(16, 16)] = acc1 * (1.0 / CTX)
        return carry

    lax.fori_loop(0, B_PER_W, row_body, 0)

    pltpu.sync_copy(pooled_v, out_hbm.at[pl.ds(wid * B_PER_W, B_PER_W)])


_sc_pool = functools.partial(
    pl.kernel,
    out_type=jax.ShapeDtypeStruct((BATCH, EMBED_DIM), jnp.float32),
    mesh=plsc.VectorSubcoreMesh(core_axis_name="c", subcore_axis_name="s"),
    scratch_types=[
        pltpu.VMEM((N_CHUNKS, IDX_CHUNK), jnp.int32),
        pltpu.VMEM((IDX_PER_W, EMBED_DIM), jnp.float32),
        pltpu.VMEM((B_PER_W, EMBED_DIM), jnp.float32),
        pltpu.SemaphoreType.DMA,
    ],
    compiler_params=pltpu.CompilerParams(use_tc_tiling_on_sc=False),
)(_pool_body)


B_GRP = 256
V_BLK = 12800
N_BG = BATCH // B_GRP
N_VB = pl.cdiv(VOCAB, V_BLK)


def _proj_body(x_ref, wt_ref, b_ref, o_ref):
    o_ref[...] = (
        lax.dot_general(
            x_ref[...],
            wt_ref[...],
            (((1,), (0,)), ((), ())),
            preferred_element_type=jnp.float32,
        )
        + b_ref[...]
    )


def _projection(pooled, Wt, b2):
    return pl.pallas_call(
        _proj_body,
        grid=(N_BG, N_VB),
        in_specs=[
            pl.BlockSpec((B_GRP, EMBED_DIM), lambda g, v: (g, 0)),
            pl.BlockSpec((EMBED_DIM, V_BLK), lambda g, v: (0, v)),
            pl.BlockSpec((1, V_BLK), lambda g, v: (0, v)),
        ],
        out_specs=pl.BlockSpec((B_GRP, V_BLK), lambda g, v: (g, v)),
        out_shape=jax.ShapeDtypeStruct((BATCH, VOCAB), jnp.float32),
        compiler_params=pltpu.CompilerParams(
            vmem_limit_bytes=110 * 1024 * 1024,
        ),
    )(pooled, Wt, b2)


def kernel(inputs, emb, W, b):
    idx = inputs.astype(jnp.int32).reshape(NW, N_CHUNKS, IDX_CHUNK)
    pooled = _sc_pool(idx, emb)
    return _projection(pooled, W.T, b.reshape(1, VOCAB))
